# Initial kernel scaffold; baseline (speedup 1.0000x reference)
import jax
import jax.numpy as jnp
from jax.experimental import pallas as pl


def kernel(x, coords, batch, params):
    raise NotImplementedError("write your pallas kernel here")


# speedup vs baseline: 2.2009x; 2.2009x over previous
import jax
import jax.numpy as jnp
import numpy as np
from jax import lax
from jax.experimental import pallas as pl

BLOCK = 64
NUM_HEADS = 8
H_DIM = 64
NWPD = 8
NSEG = 50
G = 8


def _pad0(a, m, value):
    pad = (-a.shape[0]) % m
    if pad == 0:
        return a
    pw = [(0, pad)] + [(0, 0)] * (a.ndim - 1)
    return jnp.pad(a, pw, constant_values=value)


def _ln(x, g, b):
    mu = jnp.mean(x, -1, keepdims=True)
    var = jnp.var(x, -1, keepdims=True)
    return (x - mu) / jnp.sqrt(var + 1e-5) * g + b


def _attn_kernel(xs_ref, cs_ref, wq_ref, wk_ref, wv_ref, wo_ref, pw_ref,
                 pb_ref, o_ref):
    xs = xs_ref[...]
    xf = xs.reshape(G * BLOCK, H_DIM)
    cs = cs_ref[...]
    d = cs[:, :, None] - cs[:, None, :]
    pw = pw_ref[...]
    pb = pb_ref[...]
    acc = jnp.zeros((G * BLOCK, H_DIM), jnp.float32)
    for h in range(NUM_HEADS):
        sl = slice(h * H_DIM, (h + 1) * H_DIM)
        q = jnp.dot(xf, wq_ref[sl, :], preferred_element_type=jnp.float32)
        k = jnp.dot(xf, wk_ref[sl, :], preferred_element_type=jnp.float32)
        v = jnp.dot(xf, wv_ref[sl, :], preferred_element_type=jnp.float32)
        qb = q.reshape(G, BLOCK, H_DIM)
        kb = k.reshape(G, BLOCK, H_DIM)
        vb = v.reshape(G, BLOCK, H_DIM)
        logits = lax.dot_general(
            qb, kb, (((2,), (2,)), ((0,), (0,))),
            preferred_element_type=jnp.float32) * (1.0 / 8.0)
        b = jnp.full_like(d, pw[NWPD - 1, h])
        for p in range(NWPD - 2, -1, -1):
            b = b * d + pw[p, h]
        logits = logits + b * d + pb[0, h]
        logits = logits - jnp.max(logits, axis=-1, keepdims=True)
        e = jnp.exp(logits)
        a = e / jnp.sum(e, axis=-1, keepdims=True)
        o = lax.dot_general(
            a, vb, (((2,), (1,)), ((0,), (0,))),
            preferred_element_type=jnp.float32)
        acc = acc + jnp.dot(o.reshape(G * BLOCK, H_DIM), wo_ref[sl, :],
                            preferred_element_type=jnp.float32)
    o_ref[...] = acc.reshape(G, BLOCK, H_DIM)


def _block_attn(xs, cs, wq_hm, wk_hm, wv_hm, wo, pw, pb):
    nbp = xs.shape[0] // BLOCK
    xs3 = xs.reshape(nbp, BLOCK, H_DIM)
    cs2 = cs.reshape(nbp, BLOCK)
    full = lambda shape: pl.BlockSpec(shape, lambda i: (0,) * len(shape))
    out = pl.pallas_call(
        _attn_kernel,
        grid=(nbp // G,),
        in_specs=[
            pl.BlockSpec((G, BLOCK, H_DIM), lambda i: (i, 0, 0)),
            pl.BlockSpec((G, BLOCK), lambda i: (i, 0)),
            full((NUM_HEADS * H_DIM, H_DIM)),
            full((NUM_HEADS * H_DIM, H_DIM)),
            full((NUM_HEADS * H_DIM, H_DIM)),
            full((NUM_HEADS * H_DIM, H_DIM)),
            full((NWPD, NUM_HEADS)),
            full((1, NUM_HEADS)),
        ],
        out_specs=pl.BlockSpec((G, BLOCK, H_DIM), lambda i: (i, 0, 0)),
        out_shape=jax.ShapeDtypeStruct((nbp, BLOCK, H_DIM), jnp.float32),
    )(xs3, cs2, wq_hm, wk_hm, wv_hm, wo, pw, pb)
    return out.reshape(nbp * BLOCK, H_DIM)


def _head_major(w):
    return w.reshape(H_DIM, NUM_HEADS, H_DIM).transpose(1, 0, 2).reshape(
        NUM_HEADS * H_DIM, H_DIM)


def kernel(x, coords, batch, params):
    raw = x.shape[0]
    xp = _pad0(x, BLOCK, 0.0)
    c_inf = _pad0(coords, BLOCK, float(np.inf))
    order_eta = jnp.argsort(c_inf[:, 0])
    order_phi = jnp.argsort(c_inf[:, 1])
    c0 = _pad0(coords, BLOCK, 0.0)
    npts = xp.shape[0]
    npg = ((npts // BLOCK + G - 1) // G) * G * BLOCK

    h = jax.nn.relu(xp @ params['fe_W1'] + params['fe_b1']) @ params['fe_W2'] \
        + params['fe_b2']
    all_h = [h]
    for lp in params['layers']:
        wq_hm = _head_major(lp['Wq'])
        wk_hm = _head_major(lp['Wk'])
        wv_hm = _head_major(lp['Wv'])
        pw = lp['rpe_W'].reshape(NWPD, NUM_HEADS, H_DIM).mean(-1)
        pb = lp['rpe_b'].reshape(NUM_HEADS, H_DIM).mean(-1).reshape(1, -1)
        xn = _ln(h, lp['ln1_g'], lp['ln1_b'])
        outs = []
        for ax, order in ((0, order_eta), (1, order_phi)):
            xs = _pad0(xn[order], npg, 0.0)[:npg]
            cs = _pad0(c0[order, ax], npg, 0.0)[:npg]
            os_ = _block_attn(xs, cs, wq_hm, wk_hm, wv_hm, lp['Wo'], pw, pb)
            outs.append(jnp.zeros((npts, H_DIM), jnp.float32)
                        .at[order].set(os_[:npts]))
        h = h + 0.5 * (outs[0] + outs[1])
        xn2 = _ln(h, lp['ln2_g'], lp['ln2_b'])
        ff = jax.nn.relu(xn2 @ lp['ff_W1'] + lp['ff_b1']) @ lp['ff_W2'] \
            + lp['ff_b2']
        h = h + ff
        all_h.append(h)

    enc = jnp.tanh(jnp.concatenate(all_h, -1) @ params['W_cat'])
    m = enc
    for i in range(4):
        m = jnp.tanh(_ln(m @ params['m_W'][i] + params['m_b'][i],
                         params['m_g'][i], params['m_bt'][i]))
    m = m @ params['m_W5'] + params['m_b5']
    out = enc + m
    out = out[:raw]
    sums = jax.ops.segment_sum(out, batch, num_segments=NSEG)
    cnts = jax.ops.segment_sum(jnp.ones((raw,), jnp.float32), batch,
                               num_segments=NSEG)
    pooled = sums / jnp.clip(cnts, 1.0)[:, None]
    return pooled @ params['out_W'] + params['out_b']


# speedup vs baseline: 2.6080x; 1.1849x over previous
import jax
import jax.numpy as jnp
import numpy as np
from jax import lax
from jax.experimental import pallas as pl

BLOCK = 64
NUM_HEADS = 8
H_DIM = 64
NWPD = 8
NSEG = 50
G = 8
R = 512


def _pad0(a, m, value):
    pad = (-a.shape[0]) % m
    if pad == 0:
        return a
    pw = [(0, pad)] + [(0, 0)] * (a.ndim - 1)
    return jnp.pad(a, pw, constant_values=value)


def _lnk(x, g, b):
    mu = jnp.mean(x, -1, keepdims=True)
    var = jnp.mean((x - mu) ** 2, -1, keepdims=True)
    return (x - mu) / jnp.sqrt(var + 1e-5) * g + b


def _full(shape):
    return pl.BlockSpec(shape, lambda i: (0,) * len(shape))


def _enc_kernel(x_ref, w1_ref, b1_ref, w2_ref, b2_ref, o_ref):
    h = jnp.maximum(
        jnp.dot(x_ref[...], w1_ref[...], preferred_element_type=jnp.float32)
        + b1_ref[...], 0.0)
    o_ref[...] = jnp.dot(h, w2_ref[...],
                         preferred_element_type=jnp.float32) + b2_ref[...]


def _encoder(x, w1, b1, w2, b2):
    n = x.shape[0]
    return pl.pallas_call(
        _enc_kernel,
        grid=(n // R,),
        in_specs=[pl.BlockSpec((R, 16), lambda i: (i, 0)),
                  _full((16, H_DIM)), _full((1, H_DIM)),
                  _full((H_DIM, H_DIM)), _full((1, H_DIM))],
        out_specs=pl.BlockSpec((R, H_DIM), lambda i: (i, 0)),
        out_shape=jax.ShapeDtypeStruct((n, H_DIM), jnp.float32),
    )(x, w1, b1.reshape(1, -1), w2, b2.reshape(1, -1))


def _attn_kernel(hs_ref, cs_ref, g_ref, b_ref, wq_ref, wk_ref, wv_ref,
                 wo_ref, pw_ref, pb_ref, o_ref):
    hb = hs_ref[...].reshape(G * BLOCK, H_DIM)
    xf = _lnk(hb, g_ref[...], b_ref[...])
    cs = cs_ref[...]
    d = cs[:, :, None] - cs[:, None, :]
    pw = pw_ref[...]
    pb = pb_ref[...]
    acc = jnp.zeros((G * BLOCK, H_DIM), jnp.float32)
    for h in range(NUM_HEADS):
        sl = slice(h * H_DIM, (h + 1) * H_DIM)
        q = jnp.dot(xf, wq_ref[sl, :], preferred_element_type=jnp.float32)
        k = jnp.dot(xf, wk_ref[sl, :], preferred_element_type=jnp.float32)
        v = jnp.dot(xf, wv_ref[sl, :], preferred_element_type=jnp.float32)
        qb = q.reshape(G, BLOCK, H_DIM)
        kb = k.reshape(G, BLOCK, H_DIM)
        vb = v.reshape(G, BLOCK, H_DIM)
        logits = lax.dot_general(
            qb, kb, (((2,), (2,)), ((0,), (0,))),
            preferred_element_type=jnp.float32) * (1.0 / 8.0)
        b = jnp.full_like(d, pw[NWPD - 1, h])
        for p in range(NWPD - 2, -1, -1):
            b = b * d + pw[p, h]
        logits = logits + b * d + pb[0, h]
        logits = logits - jnp.max(logits, axis=-1, keepdims=True)
        e = jnp.exp(logits)
        a = e / jnp.sum(e, axis=-1, keepdims=True)
        o = lax.dot_general(
            a, vb, (((2,), (1,)), ((0,), (0,))),
            preferred_element_type=jnp.float32)
        acc = acc + jnp.dot(o.reshape(G * BLOCK, H_DIM), wo_ref[sl, :],
                            preferred_element_type=jnp.float32)
    o_ref[...] = acc.reshape(G, BLOCK, H_DIM)


def _block_attn(hs, cs, lg, lb, wq_hm, wk_hm, wv_hm, wo, pw, pb):
    nbp = hs.shape[0] // BLOCK
    hs3 = hs.reshape(nbp, BLOCK, H_DIM)
    cs2 = cs.reshape(nbp, BLOCK)
    out = pl.pallas_call(
        _attn_kernel,
        grid=(nbp // G,),
        in_specs=[
            pl.BlockSpec((G, BLOCK, H_DIM), lambda i: (i, 0, 0)),
            pl.BlockSpec((G, BLOCK), lambda i: (i, 0)),
            _full((1, H_DIM)), _full((1, H_DIM)),
            _full((NUM_HEADS * H_DIM, H_DIM)),
            _full((NUM_HEADS * H_DIM, H_DIM)),
            _full((NUM_HEADS * H_DIM, H_DIM)),
            _full((NUM_HEADS * H_DIM, H_DIM)),
            _full((NWPD, NUM_HEADS)),
            _full((1, NUM_HEADS)),
        ],
        out_specs=pl.BlockSpec((G, BLOCK, H_DIM), lambda i: (i, 0, 0)),
        out_shape=jax.ShapeDtypeStruct((nbp, BLOCK, H_DIM), jnp.float32),
    )(hs3, cs2, lg.reshape(1, -1), lb.reshape(1, -1),
      wq_hm, wk_hm, wv_hm, wo, pw, pb)
    return out.reshape(nbp * BLOCK, H_DIM)


def _ffn_kernel(h_ref, oe_ref, op_ref, g2_ref, b2_ref, w1_ref, bf1_ref,
                w2_ref, bf2_ref, o_ref):
    h = h_ref[...] + 0.5 * (oe_ref[...] + op_ref[...])
    xn2 = _lnk(h, g2_ref[...], b2_ref[...])
    ff = jnp.maximum(
        jnp.dot(xn2, w1_ref[...], preferred_element_type=jnp.float32)
        + bf1_ref[...], 0.0)
    ff = jnp.dot(ff, w2_ref[...],
                 preferred_element_type=jnp.float32) + bf2_ref[...]
    o_ref[...] = h + ff


def _ffn(h, oe, op, lp):
    n = h.shape[0]
    row = pl.BlockSpec((R, H_DIM), lambda i: (i, 0))
    return pl.pallas_call(
        _ffn_kernel,
        grid=(n // R,),
        in_specs=[row, row, row,
                  _full((1, H_DIM)), _full((1, H_DIM)),
                  _full((H_DIM, H_DIM)), _full((1, H_DIM)),
                  _full((H_DIM, H_DIM)), _full((1, H_DIM))],
        out_specs=row,
        out_shape=jax.ShapeDtypeStruct((n, H_DIM), jnp.float32),
    )(h, oe, op, lp['ln2_g'].reshape(1, -1), lp['ln2_b'].reshape(1, -1),
      lp['ff_W1'], lp['ff_b1'].reshape(1, -1),
      lp['ff_W2'], lp['ff_b2'].reshape(1, -1))


def _head_kernel(h0_ref, h1_ref, h2_ref, bt_ref, wcat_ref, mw0_ref, mw_ref,
                 mb_ref, mg_ref, mbt_ref, mw5_ref, mb5_ref, outw_ref,
                 outb_ref, acc_ref, res_ref):
    i = pl.program_id(0)
    ng = pl.num_programs(0)
    enc_in = jnp.concatenate([h0_ref[...], h1_ref[...], h2_ref[...]], axis=1)
    enc = jnp.tanh(jnp.dot(enc_in, wcat_ref[...],
                           preferred_element_type=jnp.float32))
    m = jnp.dot(enc, mw0_ref[...],
                preferred_element_type=jnp.float32) + mb_ref[0:1, :]
    m = jnp.tanh(_lnk(m, mg_ref[0:1, :], mbt_ref[0:1, :]))
    for j in range(1, 4):
        w = mw_ref[(j - 1) * 256:j * 256, :]
        m = jnp.dot(m, w, preferred_element_type=jnp.float32) + mb_ref[j:j+1, :]
        m = jnp.tanh(_lnk(m, mg_ref[j:j+1, :], mbt_ref[j:j+1, :]))
    m = jnp.dot(m, mw5_ref[...],
                preferred_element_type=jnp.float32) + mb5_ref[...]
    out = enc + m
    vals = jnp.concatenate([out, jnp.ones((R, 32), jnp.float32)], axis=1)
    lanes = lax.broadcasted_iota(jnp.int32, (R, 64), 1).astype(jnp.float32)
    onehot = (bt_ref[...] == lanes).astype(jnp.float32)
    upd = lax.dot_general(onehot, vals, (((0,), (0,)), ((), ())),
                          preferred_element_type=jnp.float32)

    @pl.when(i == 0)
    def _init():
        acc_ref[...] = jnp.zeros_like(acc_ref)

    acc_ref[...] += upd

    @pl.when(i == ng - 1)
    def _final():
        a = acc_ref[...]
        pooled = a[:, :32] / jnp.maximum(a[:, 32:33], 1.0)
        res_ref[...] = jnp.dot(pooled, outw_ref[...],
                               preferred_element_type=jnp.float32) \
            + outb_ref[...]


def _head(h0, h1, h2, bt, params):
    n = h0.shape[0]
    row = pl.BlockSpec((R, H_DIM), lambda i: (i, 0))
    mw = jnp.concatenate(params['m_W'][1:], axis=0)
    mb = jnp.stack(params['m_b'])
    mg = jnp.stack(params['m_g'])
    mbt = jnp.stack(params['m_bt'])
    acc, res = pl.pallas_call(
        _head_kernel,
        grid=(n // R,),
        in_specs=[row, row, row,
                  pl.BlockSpec((R, 1), lambda i: (i, 0)),
                  _full((3 * H_DIM, 32)),
                  _full((32, 256)), _full((768, 256)),
                  _full((4, 256)), _full((4, 256)), _full((4, 256)),
                  _full((256, 32)), _full((1, 32)),
                  _full((32, 1)), _full((1, 1))],
        out_specs=[_full((64, 64)), _full((64, 1))],
        out_shape=[jax.ShapeDtypeStruct((64, 64), jnp.float32),
                   jax.ShapeDtypeStruct((64, 1), jnp.float32)],
    )(h0, h1, h2, bt, params['W_cat'], params['m_W'][0], mw, mb, mg, mbt,
      params['m_W5'], params['m_b5'].reshape(1, -1),
      params['out_W'], params['out_b'].reshape(1, -1))
    return res[:NSEG, :]


def _head_major(w):
    return w.reshape(H_DIM, NUM_HEADS, H_DIM).transpose(1, 0, 2).reshape(
        NUM_HEADS * H_DIM, H_DIM)


def kernel(x, coords, batch, params):
    raw = x.shape[0]
    xp = _pad0(x, BLOCK, 0.0)
    c_inf = _pad0(coords, BLOCK, float(np.inf))
    order_eta = jnp.argsort(c_inf[:, 0])
    order_phi = jnp.argsort(c_inf[:, 1])
    c0 = _pad0(coords, BLOCK, 0.0)
    npts = xp.shape[0]
    npg = ((npts // BLOCK + G - 1) // G) * G * BLOCK

    inv_eta = jnp.zeros((npts,), jnp.int32).at[order_eta].set(
        jnp.arange(npts, dtype=jnp.int32))
    rel = inv_eta[order_phi]
    rel_pad = jnp.concatenate(
        [rel, jnp.arange(npts, npg, dtype=jnp.int32)])
    x_home = _pad0(xp[order_eta], npg, 0.0)[:npg]
    c_eta = _pad0(c0[order_eta, 0], npg, 0.0)[:npg]
    c_phi = _pad0(c0[order_phi, 1], npg, 0.0)[:npg]
    batch_home = _pad0(
        _pad0(batch, BLOCK, NSEG)[order_eta], npg, NSEG
    )[:npg].astype(jnp.float32).reshape(-1, 1)

    h = _encoder(x_home, params['fe_W1'], params['fe_b1'],
                 params['fe_W2'], params['fe_b2'])
    all_h = [h]
    for lp in params['layers']:
        wq_hm = _head_major(lp['Wq'])
        wk_hm = _head_major(lp['Wk'])
        wv_hm = _head_major(lp['Wv'])
        pw = lp['rpe_W'].reshape(NWPD, NUM_HEADS, H_DIM).mean(-1)
        pb = lp['rpe_b'].reshape(NUM_HEADS, H_DIM).mean(-1).reshape(1, -1)
        oe = _block_attn(h, c_eta, lp['ln1_g'], lp['ln1_b'],
                         wq_hm, wk_hm, wv_hm, lp['Wo'], pw, pb)
        hs_phi = h[rel_pad]
        os_phi = _block_attn(hs_phi, c_phi, lp['ln1_g'], lp['ln1_b'],
                             wq_hm, wk_hm, wv_hm, lp['Wo'], pw, pb)
        op = jnp.zeros((npg, H_DIM), jnp.float32).at[rel_pad].set(os_phi)
        h = _ffn(h, oe, op, lp)
        all_h.append(h)

    return _head(all_h[0], all_h[1], all_h[2], batch_home, params)
